# baseline (speedup 1.0000x reference)
import jax
import jax.numpy as jnp
from jax.experimental import pallas as pl
from jax.experimental.pallas import tpu as pltpu

NUM_EMBEDDINGS = 512
EMBEDDING_DIM = 32
BETA = 0.25
ROW_TILE = 2048
BATCH_TILE = ROW_TILE // 1024


def _vq_body(z_ref, cb_ref, cn_ref, perp_ref, loss_ref, q_ref,
             counts_acc, dsum_acc, *, n_tiles, n_rows):
    i = pl.program_id(0)
    z = z_ref[...].reshape(ROW_TILE, EMBEDDING_DIM)
    cb = cb_ref[...]
    dot2 = jnp.dot(z + z, cb, preferred_element_type=jnp.float32)
    znorm = jnp.sum(z * z, axis=1, keepdims=True)
    d = znorm - dot2 + cn_ref[...]
    dmin = jnp.min(d, axis=1, keepdims=True)
    k_iota = jax.lax.broadcasted_iota(jnp.int32, d.shape, 1).astype(jnp.float32)
    idxf = jnp.min(jnp.where(d == dmin, k_iota, float(NUM_EMBEDDINGS)),
                   axis=1, keepdims=True)
    onehot = (k_iota == idxf).astype(jnp.float32)

    @pl.when(i == 0)
    def _init():
        counts_acc[...] = jnp.zeros_like(counts_acc)
        dsum_acc[...] = jnp.zeros_like(dsum_acc)

    counts_acc[...] += jnp.sum(onehot, axis=0, keepdims=True)
    dsum_acc[...] += jnp.full((1, 128), jnp.sum(dmin), jnp.float32)
    q = jax.lax.dot_general(onehot, cb, (((1,), (1,)), ((), ())),
                            preferred_element_type=jnp.float32)
    q_ref[...] = q.reshape(BATCH_TILE, 1024, EMBEDDING_DIM)

    @pl.when(i == n_tiles - 1)
    def _finalize():
        avg = counts_acc[...] * (1.0 / n_rows)
        perp = jnp.exp(-jnp.sum(avg * jnp.log(avg + 1e-10)))
        perp_ref[...] = jnp.full((1, 1), perp, jnp.float32)
        loss_ref[...] = dsum_acc[:, :1] * (BETA / (n_rows * EMBEDDING_DIM))


def kernel(inputs, codebook, ema_cs_hidden, ema_dw_hidden, counter, training):
    batch, hw, dim = inputs.shape
    n_rows = batch * hw
    n_tiles = n_rows // ROW_TILE
    cnorm = jnp.sum(codebook * codebook, axis=0, keepdims=True)

    body = lambda *refs: _vq_body(*refs, n_tiles=n_tiles, n_rows=n_rows)
    perp2, loss2, ste = pl.pallas_call(
        body,
        grid=(n_tiles,),
        in_specs=[
            pl.BlockSpec((BATCH_TILE, hw, dim), lambda i: (i, 0, 0)),
            pl.BlockSpec((dim, NUM_EMBEDDINGS), lambda i: (0, 0)),
            pl.BlockSpec((1, NUM_EMBEDDINGS), lambda i: (0, 0)),
        ],
        out_specs=[
            pl.BlockSpec((1, 1), lambda i: (0, 0)),
            pl.BlockSpec((1, 1), lambda i: (0, 0)),
            pl.BlockSpec((BATCH_TILE, hw, dim), lambda i: (i, 0, 0)),
        ],
        out_shape=[
            jax.ShapeDtypeStruct((1, 1), jnp.float32),
            jax.ShapeDtypeStruct((1, 1), jnp.float32),
            jax.ShapeDtypeStruct((batch, hw, dim), jnp.float32),
        ],
        scratch_shapes=[
            pltpu.VMEM((1, NUM_EMBEDDINGS), jnp.float32),
            pltpu.VMEM((1, 128), jnp.float32),
        ],
        compiler_params=pltpu.CompilerParams(
            dimension_semantics=("arbitrary",),
        ),
    )(inputs, codebook, cnorm)

    return ste, perp2.reshape(()), loss2.reshape(())
